# Initial kernel scaffold; baseline (speedup 1.0000x reference)
import jax
import jax.numpy as jnp
from jax.experimental import pallas as pl


def kernel(xyz, delta, index):
    raise NotImplementedError("write your pallas kernel here")


# speedup vs baseline: 8.4824x; 8.4824x over previous
import functools

import jax
import jax.numpy as jnp
from jax import lax
from jax.experimental import pallas as pl
from jax.experimental.pallas import tpu as pltpu
from jax.experimental.pallas import tpu_sc as plsc

N = 1_000_000
D = 8
C = 2000
NCH = N // C
NW = 32
GB = [128] * 15 + [80]
GOFF = [sum(GB[:i]) for i in range(len(GB))]


def _rsqrt(x):
    i = plsc.bitcast(x, jnp.int32)
    y = plsc.bitcast(jnp.int32(0x5F3759DF) - (i >> 1), jnp.float32)
    for _ in range(3):
        y = y * (1.5 - 0.5 * x * y * y)
    return y


def _sc_body(idx_hbm, table_hbm, xyz_out, rot_out,
             idx_v, acc_v, xyz_v, rot_v, sem):
    cid = lax.axis_index("c")
    sid = lax.axis_index("s")
    w = sid * 2 + cid
    nw_chunks = (NCH - w + NW - 1) // NW

    def do_chunk(i, carry):
        base = (w + i * NW) * C
        for l in range(3):
            pltpu.sync_copy(idx_hbm.at[pl.ds(l * N + base, C)],
                            idx_v.at[pl.ds(l * C, C)])
        for l in range(3):
            descs = []
            for g, off in zip(GB, GOFF):
                descs.append(pltpu.async_copy(
                    table_hbm.at[idx_v.at[pl.ds(l * C + off, g)]],
                    acc_v.at[pl.ds(off, g), :],
                    sem, add=(l > 0)))
            for d in descs:
                d.wait()

        def norm16(j, carry2):
            rows = j * 16 + lax.iota(jnp.int32, 16)
            cols = [plsc.load_gather(acc_v, [rows, jnp.full((16,), c, jnp.int32)])
                    for c in range(7)]
            q0, q1, q2, q3 = cols[3], cols[4], cols[5], cols[6]
            n2 = q0 * q0 + q1 * q1 + q2 * q2 + q3 * q3
            norm = n2 * _rsqrt(n2)
            inv = 1.0 / jnp.maximum(norm, 1e-12)
            for c in range(3):
                plsc.store_scatter(xyz_v, [rows * 3 + c], cols[c])
            for k in range(4):
                plsc.store_scatter(rot_v, [rows * 4 + k], cols[3 + k] * inv)
            return carry2

        lax.fori_loop(0, C // 16, norm16, 0)
        pltpu.sync_copy(xyz_v, xyz_out.at[pl.ds(base * 3, C * 3)])
        pltpu.sync_copy(rot_v, rot_out.at[pl.ds(base * 4, C * 4)])
        return carry

    lax.fori_loop(0, nw_chunks, do_chunk, 0)


@jax.jit
def _deform(table, idx_flat):
    mesh = plsc.VectorSubcoreMesh(core_axis_name="c", subcore_axis_name="s")
    f = pl.kernel(
        _sc_body,
        out_type=(jax.ShapeDtypeStruct((N * 3,), jnp.float32),
                  jax.ShapeDtypeStruct((N * 4,), jnp.float32)),
        mesh=mesh,
        scratch_types=[
            pltpu.VMEM((3 * C,), jnp.int32),
            pltpu.VMEM((C, D), jnp.float32),
            pltpu.VMEM((C * 3,), jnp.float32),
            pltpu.VMEM((C * 4,), jnp.float32),
            pltpu.SemaphoreType.DMA,
        ],
        compiler_params=pltpu.CompilerParams(use_tc_tiling_on_sc=False, needs_layout_passes=False),
    )
    return f(idx_flat, table)


def kernel(xyz, delta, index):
    table = jnp.pad(delta, ((0, 0), (0, D - delta.shape[1])))
    xyz_flat, rot_flat = _deform(table, index.reshape(-1))
    return xyz_flat.reshape(N, 3), rot_flat.reshape(N, 4)


# speedup vs baseline: 9.0344x; 1.0651x over previous
import jax
import jax.numpy as jnp
from jax import lax
from jax.experimental import pallas as pl
from jax.experimental.pallas import tpu as pltpu
from jax.experimental.pallas import tpu_sc as plsc

N = 1_000_000
D = 8
C = 2000
NCH = N // C
NW = 32
GB = [128] * 15 + [80]
GOFF = [sum(GB[:i]) for i in range(len(GB))]


def _rsqrt(x):
    i = plsc.bitcast(x, jnp.int32)
    y = plsc.bitcast(jnp.int32(0x5F3759DF) - (i >> 1), jnp.float32)
    for _ in range(2):
        y = y * (1.5 - 0.5 * x * y * y)
    return y


def _sc_body(idx_hbm, table_hbm, xyz_out, rot_out,
             idx_v, acc_v, xyz_v, rot_v, sem_g, sem_o):
    cid = lax.axis_index("c")
    sid = lax.axis_index("s")
    w = sid * 2 + cid
    nw_chunks = (NCH - w + NW - 1) // NW

    def gather_descs(k, b):
        base = (w + k * NW) * C
        out = []
        for l in range(3):
            for g, off in zip(GB, GOFF):
                out.append(pltpu.async_copy(
                    table_hbm.at[idx_v.at[b, pl.ds(l * C + off, g)]],
                    acc_v.at[b, pl.ds(off, g), :],
                    sem_g, add=(l > 0)))
        return out

    def issue(k, b):
        base = (w + k * NW) * C
        for l in range(3):
            pltpu.sync_copy(idx_hbm.at[pl.ds(l * N + base, C)],
                            idx_v.at[b, pl.ds(l * C, C)])
        gather_descs(k, b)

    def drain_gathers(b):
        for l in range(3):
            for g, off in zip(GB, GOFF):
                pltpu.make_async_copy(
                    table_hbm.at[idx_v.at[b, pl.ds(l * C + off, g)]],
                    acc_v.at[b, pl.ds(off, g), :],
                    sem_g).wait()

    def out_descs(k, b):
        base = (w + k * NW) * C
        return [
            pltpu.async_copy(xyz_v.at[b], xyz_out.at[pl.ds(base * 3, C * 3)],
                             sem_o),
            pltpu.async_copy(rot_v.at[b], rot_out.at[pl.ds(base * 4, C * 4)],
                             sem_o),
        ]

    def drain_out(b):
        pltpu.make_async_copy(xyz_v.at[b], xyz_out.at[pl.ds(0, C * 3)],
                              sem_o).wait()
        pltpu.make_async_copy(rot_v.at[b], rot_out.at[pl.ds(0, C * 4)],
                              sem_o).wait()

    def norm_chunk(b):
        accb = acc_v.at[b]
        xyzb = xyz_v.at[b]
        rotb = rot_v.at[b]

        def norm16(j, carry2):
            rows = j * 16 + lax.iota(jnp.int32, 16)
            cols = [plsc.load_gather(accb, [rows, jnp.full((16,), c, jnp.int32)])
                    for c in range(7)]
            q0, q1, q2, q3 = cols[3], cols[4], cols[5], cols[6]
            n2 = q0 * q0 + q1 * q1 + q2 * q2 + q3 * q3
            y = _rsqrt(n2)
            inv = jnp.where(n2 * y >= 1e-12, y, 1e12)
            for c in range(3):
                plsc.store_scatter(xyzb, [rows * 3 + c], cols[c])
            for k in range(4):
                plsc.store_scatter(rotb, [rows * 4 + k], cols[3 + k] * inv)
            return carry2

        lax.fori_loop(0, C // 16, norm16, 0)

    issue(0, 0)

    def do_chunk(i, carry):
        p = lax.rem(i, 2)
        drain_gathers(p)

        @pl.when(i + 1 < nw_chunks)
        def _():
            issue(i + 1, 1 - p)

        @pl.when(i >= 2)
        def _():
            drain_out(p)

        norm_chunk(p)
        out_descs(i, p)
        return carry

    lax.fori_loop(0, nw_chunks, do_chunk, 0)
    drain_out(lax.rem(nw_chunks - 1, 2))

    @pl.when(nw_chunks >= 2)
    def _():
        drain_out(lax.rem(nw_chunks, 2))


@jax.jit
def _deform(table, idx_flat):
    mesh = plsc.VectorSubcoreMesh(core_axis_name="c", subcore_axis_name="s")
    f = pl.kernel(
        _sc_body,
        out_type=(jax.ShapeDtypeStruct((N * 3,), jnp.float32),
                  jax.ShapeDtypeStruct((N * 4,), jnp.float32)),
        mesh=mesh,
        scratch_types=[
            pltpu.VMEM((2, 3 * C), jnp.int32),
            pltpu.VMEM((2, C, D), jnp.float32),
            pltpu.VMEM((2, C * 3), jnp.float32),
            pltpu.VMEM((2, C * 4), jnp.float32),
            pltpu.SemaphoreType.DMA,
            pltpu.SemaphoreType.DMA,
        ],
        compiler_params=pltpu.CompilerParams(use_tc_tiling_on_sc=False,
                                             needs_layout_passes=False),
    )
    return f(idx_flat, table)


def kernel(xyz, delta, index):
    table = jnp.pad(delta, ((0, 0), (0, D - delta.shape[1])))
    xyz_flat, rot_flat = _deform(table, index.reshape(-1))
    return xyz_flat.reshape(N, 3), rot_flat.reshape(N, 4)


# speedup vs baseline: 9.0790x; 1.0049x over previous
import jax
import jax.numpy as jnp
from jax import lax
from jax.experimental import pallas as pl
from jax.experimental.pallas import tpu as pltpu
from jax.experimental.pallas import tpu_sc as plsc

N = 1_000_000
D = 8
C = 800
NCH = N // C
NW = 32
GB = [128] * 6 + [32]
GOFF = [sum(GB[:i]) for i in range(len(GB))]
S0, S1, S2 = 59, 29, 14
OFF2 = S0**3 + S1**3
NL2 = S2**3
SPLIT_BLK = 16384


def _rsqrt(x):
    i = plsc.bitcast(x, jnp.int32)
    y = plsc.bitcast(jnp.int32(0x5F3759DF) - (i >> 1), jnp.float32)
    for _ in range(2):
        y = y * (1.5 - 0.5 * x * y * y)
    return y


def _split_body(idx_ref, o0, o1, o2):
    o0[...] = idx_ref[0, :]
    o1[...] = idx_ref[1, :]
    o2[...] = idx_ref[2, :]


def _split_index(index):
    return pl.pallas_call(
        _split_body,
        grid=(pl.cdiv(N, SPLIT_BLK),),
        in_specs=[pl.BlockSpec((3, SPLIT_BLK), lambda i: (0, i))],
        out_specs=[pl.BlockSpec((SPLIT_BLK,), lambda i: (i,))] * 3,
        out_shape=[jax.ShapeDtypeStruct((N,), jnp.int32)] * 3,
    )(index)


def _sc_body(idx0_hbm, idx1_hbm, idx2_hbm, table_hbm, xyz_out, rot_out,
             idx_v, acc_v, l2_v, xyz_v, rot_v,
             sem_l0, sem_l1, sem_o0, sem_o1, sem_o2):
    cid = lax.axis_index("c")
    sid = lax.axis_index("s")
    w = sid * 2 + cid
    nw_chunks = (NCH - w + NW - 1) // NW
    sem_o = [sem_o0, sem_o1, sem_o2]
    idx_l = [idx0_hbm, idx1_hbm, idx2_hbm]

    pltpu.sync_copy(table_hbm.at[pl.ds(OFF2, NL2), :], l2_v)

    def issue_a(k, b):
        base = (w + k * NW) * C
        for l in range(3):
            pltpu.sync_copy(idx_l[l].at[pl.ds(base, C)],
                            idx_v.at[b, pl.ds(l * C, C)])
        for g, off in zip(GB, GOFF):
            pltpu.async_copy(
                table_hbm.at[idx_v.at[b, pl.ds(off, g)]],
                acc_v.at[b, pl.ds(off, g), :],
                sem_l0)

    def drain_l0(b):
        for g, off in zip(GB, GOFF):
            pltpu.make_async_copy(
                table_hbm.at[idx_v.at[b, pl.ds(off, g)]],
                acc_v.at[b, pl.ds(off, g), :],
                sem_l0).wait()

    def issue_b(b):
        for g, off in zip(GB, GOFF):
            pltpu.async_copy(
                table_hbm.at[idx_v.at[b, pl.ds(C + off, g)]],
                acc_v.at[b, pl.ds(off, g), :],
                sem_l1, add=True)

    def drain_l1(b):
        for g, off in zip(GB, GOFF):
            pltpu.make_async_copy(
                table_hbm.at[idx_v.at[b, pl.ds(C + off, g)]],
                acc_v.at[b, pl.ds(off, g), :],
                sem_l1).wait()

    def issue_out(k, b):
        base = (w + k * NW) * C
        pltpu.async_copy(xyz_v.at[b], xyz_out.at[pl.ds(base * 3, C * 3)],
                         sem_o[b])
        pltpu.async_copy(rot_v.at[b], rot_out.at[pl.ds(base * 4, C * 4)],
                         sem_o[b])

    def drain_out(b):
        pltpu.make_async_copy(xyz_v.at[b], xyz_out.at[pl.ds(0, C * 3)],
                              sem_o[b]).wait()
        pltpu.make_async_copy(rot_v.at[b], rot_out.at[pl.ds(0, C * 4)],
                              sem_o[b]).wait()

    def norm_chunk(b):
        accb = acc_v.at[b]
        idxb = idx_v.at[b]
        xyzb = xyz_v.at[b]
        rotb = rot_v.at[b]

        def norm16(j, carry2):
            rows = j * 16 + lax.iota(jnp.int32, 16)
            i2 = idxb[pl.ds(2 * C + j * 16, 16)] - OFF2
            cols = []
            for c in range(7):
                fc = jnp.full((16,), c, jnp.int32)
                cols.append(plsc.load_gather(accb, [rows, fc])
                            + plsc.load_gather(l2_v, [i2, fc]))
            q0, q1, q2, q3 = cols[3], cols[4], cols[5], cols[6]
            n2 = q0 * q0 + q1 * q1 + q2 * q2 + q3 * q3
            y = _rsqrt(n2)
            inv = jnp.where(n2 * y >= 1e-12, y, 1e12)
            for c in range(3):
                plsc.store_scatter(xyzb, [rows * 3 + c], cols[c])
            for k in range(4):
                plsc.store_scatter(rotb, [rows * 4 + k], cols[3 + k] * inv)
            return carry2

        lax.fori_loop(0, C // 16, norm16, 0)

    issue_a(0, 0)
    issue_a(1, 1)
    drain_l0(0)
    issue_b(0)

    def step(i, bC, bB, bA):
        drain_l1(bC)

        @pl.when(i + 1 < nw_chunks)
        def _():
            drain_l0(bB)
            issue_b(bB)

        @pl.when(i + 2 < nw_chunks)
        def _():
            issue_a(i + 2, bA)

        @pl.when(i >= 3)
        def _():
            drain_out(bC)

        norm_chunk(bC)
        issue_out(i, bC)

    def triple_body(t, carry):
        i = 3 * t
        step(i, 0, 1, 2)

        @pl.when(i + 1 < nw_chunks)
        def _():
            step(i + 1, 1, 2, 0)

        @pl.when(i + 2 < nw_chunks)
        def _():
            step(i + 2, 2, 0, 1)

        return carry

    lax.fori_loop(0, (nw_chunks + 2) // 3, triple_body, 0)
    drain_out(0)
    drain_out(1)
    drain_out(2)


@jax.jit
def _deform(table, idx0, idx1, idx2):
    mesh = plsc.VectorSubcoreMesh(core_axis_name="c", subcore_axis_name="s")
    f = pl.kernel(
        _sc_body,
        out_type=(jax.ShapeDtypeStruct((N * 3,), jnp.float32),
                  jax.ShapeDtypeStruct((N * 4,), jnp.float32)),
        mesh=mesh,
        scratch_types=[
            pltpu.VMEM((3, 3 * C), jnp.int32),
            pltpu.VMEM((3, C, D), jnp.float32),
            pltpu.VMEM((NL2, D), jnp.float32),
            pltpu.VMEM((3, C * 3), jnp.float32),
            pltpu.VMEM((3, C * 4), jnp.float32),
            pltpu.SemaphoreType.DMA,
            pltpu.SemaphoreType.DMA,
            pltpu.SemaphoreType.DMA,
            pltpu.SemaphoreType.DMA,
            pltpu.SemaphoreType.DMA,
        ],
        compiler_params=pltpu.CompilerParams(use_tc_tiling_on_sc=False,
                                             needs_layout_passes=False),
    )
    return f(idx0, idx1, idx2, table)


def kernel(xyz, delta, index):
    idx0, idx1, idx2 = _split_index(index)
    table = jnp.pad(delta, ((0, 0), (0, D - delta.shape[1])))
    xyz_flat, rot_flat = _deform(table, idx0, idx1, idx2)
    return xyz_flat.reshape(N, 3), rot_flat.reshape(N, 4)


# speedup vs baseline: 32.1163x; 3.5374x over previous
import jax
import jax.numpy as jnp
from jax import lax
from jax.experimental import pallas as pl
from jax.experimental.pallas import tpu as pltpu
from jax.experimental.pallas import tpu_sc as plsc

N = 1_000_000
D = 8
C = 800
NCH = N // C
NW = 32
GB = [128] * 6 + [32]
GOFF = [sum(GB[:i]) for i in range(len(GB))]
S0, S1, S2 = 59, 29, 14
OFF2 = S0**3 + S1**3
NL2 = S2**3
NPAD = 1_000_064
SPLIT_BLK = 16384


def _rsqrt(x):
    i = plsc.bitcast(x, jnp.int32)
    y = plsc.bitcast(jnp.int32(0x5F3759DF) - (i >> 1), jnp.float32)
    for _ in range(2):
        y = y * (1.5 - 0.5 * x * y * y)
    return y


def _split_body(idx_ref, o0, o1, o2):
    o0[...] = idx_ref[0, :]
    o1[...] = idx_ref[1, :]
    o2[...] = idx_ref[2, :]


def _split_index(index):
    return pl.pallas_call(
        _split_body,
        grid=(pl.cdiv(N, SPLIT_BLK),),
        in_specs=[pl.BlockSpec((3, SPLIT_BLK), lambda i: (0, i))],
        out_specs=[pl.BlockSpec((SPLIT_BLK,), lambda i: (i,))] * 3,
        out_shape=[jax.ShapeDtypeStruct((N,), jnp.int32)] * 3,
    )(index)


def _sc_body(idx0_hbm, idx1_hbm, idx2_hbm, table_hbm, xyz_out, rot_out,
             idx_v, acc_v, l2_v, xyz_v, rot_v,
             sem_l0, sem_l1, sem_o0, sem_o1, sem_o2):
    cid = lax.axis_index("c")
    sid = lax.axis_index("s")
    w = sid * 2 + cid
    nw_chunks = (NCH - w + NW - 1) // NW
    sem_o = [sem_o0, sem_o1, sem_o2]
    idx_l = [idx0_hbm, idx1_hbm, idx2_hbm]

    pltpu.sync_copy(table_hbm.at[pl.ds(OFF2, NL2), :], l2_v)

    def issue_a(k, b):
        base = (w + k * NW) * C
        for l in range(3):
            pltpu.sync_copy(idx_l[l].at[pl.ds(base, C)],
                            idx_v.at[b, pl.ds(l * C, C)])
        for g, off in zip(GB, GOFF):
            pltpu.async_copy(
                table_hbm.at[idx_v.at[b, pl.ds(off, g)]],
                acc_v.at[b, pl.ds(off, g), :],
                sem_l0)

    def drain_l0(b):
        for g, off in zip(GB, GOFF):
            pltpu.make_async_copy(
                table_hbm.at[idx_v.at[b, pl.ds(off, g)]],
                acc_v.at[b, pl.ds(off, g), :],
                sem_l0).wait()

    def issue_b(b):
        for g, off in zip(GB, GOFF):
            pltpu.async_copy(
                table_hbm.at[idx_v.at[b, pl.ds(C + off, g)]],
                acc_v.at[b, pl.ds(off, g), :],
                sem_l1, add=True)

    def drain_l1(b):
        for g, off in zip(GB, GOFF):
            pltpu.make_async_copy(
                table_hbm.at[idx_v.at[b, pl.ds(C + off, g)]],
                acc_v.at[b, pl.ds(off, g), :],
                sem_l1).wait()

    def issue_out(k, b):
        base = (w + k * NW) * C
        pltpu.async_copy(xyz_v.at[b], xyz_out.at[:, pl.ds(base, C)], sem_o[b])
        pltpu.async_copy(rot_v.at[b], rot_out.at[:, pl.ds(base, C)], sem_o[b])

    def drain_out(b):
        pltpu.make_async_copy(xyz_v.at[b], xyz_out.at[:, pl.ds(0, C)],
                              sem_o[b]).wait()
        pltpu.make_async_copy(rot_v.at[b], rot_out.at[:, pl.ds(0, C)],
                              sem_o[b]).wait()

    def norm_chunk(b):
        accb = acc_v.at[b]
        idxb = idx_v.at[b]
        xyzb = xyz_v.at[b]
        rotb = rot_v.at[b]

        def norm16(j, carry2):
            rows = j * 16 + lax.iota(jnp.int32, 16)
            i2 = idxb[pl.ds(2 * C + j * 16, 16)] - OFF2
            cols = []
            for c in range(7):
                fc = jnp.full((16,), c, jnp.int32)
                cols.append(plsc.load_gather(accb, [rows, fc])
                            + plsc.load_gather(l2_v, [i2, fc]))
            q0, q1, q2, q3 = cols[3], cols[4], cols[5], cols[6]
            n2 = q0 * q0 + q1 * q1 + q2 * q2 + q3 * q3
            y = _rsqrt(n2)
            inv = jnp.where(n2 * y >= 1e-12, y, 1e12)
            for c in range(3):
                plsc.store_scatter(xyzb, [jnp.full((16,), c, jnp.int32), rows],
                                   cols[c])
            for k in range(4):
                plsc.store_scatter(rotb, [jnp.full((16,), k, jnp.int32), rows],
                                   cols[3 + k] * inv)
            return carry2

        lax.fori_loop(0, C // 16, norm16, 0)

    issue_a(0, 0)
    issue_a(1, 1)
    drain_l0(0)
    issue_b(0)

    def step(i, bC, bB, bA):
        drain_l1(bC)

        @pl.when(i + 1 < nw_chunks)
        def _():
            drain_l0(bB)
            issue_b(bB)

        @pl.when(i + 2 < nw_chunks)
        def _():
            issue_a(i + 2, bA)

        @pl.when(i >= 3)
        def _():
            drain_out(bC)

        norm_chunk(bC)
        issue_out(i, bC)

    def triple_body(t, carry):
        i = 3 * t
        step(i, 0, 1, 2)

        @pl.when(i + 1 < nw_chunks)
        def _():
            step(i + 1, 1, 2, 0)

        @pl.when(i + 2 < nw_chunks)
        def _():
            step(i + 2, 2, 0, 1)

        return carry

    lax.fori_loop(0, (nw_chunks + 2) // 3, triple_body, 0)
    drain_out(0)
    drain_out(1)
    drain_out(2)


@jax.jit
def _deform(table, idx0, idx1, idx2):
    mesh = plsc.VectorSubcoreMesh(core_axis_name="c", subcore_axis_name="s")
    f = pl.kernel(
        _sc_body,
        out_type=(jax.ShapeDtypeStruct((3, NPAD), jnp.float32),
                  jax.ShapeDtypeStruct((4, NPAD), jnp.float32)),
        mesh=mesh,
        scratch_types=[
            pltpu.VMEM((3, 3 * C), jnp.int32),
            pltpu.VMEM((3, C, D), jnp.float32),
            pltpu.VMEM((NL2, D), jnp.float32),
            pltpu.VMEM((3, 3, C), jnp.float32),
            pltpu.VMEM((3, 4, C), jnp.float32),
            pltpu.SemaphoreType.DMA,
            pltpu.SemaphoreType.DMA,
            pltpu.SemaphoreType.DMA,
            pltpu.SemaphoreType.DMA,
            pltpu.SemaphoreType.DMA,
        ],
        compiler_params=pltpu.CompilerParams(use_tc_tiling_on_sc=False,
                                             needs_layout_passes=False),
    )
    return f(idx0, idx1, idx2, table)


def kernel(xyz, delta, index):
    idx0, idx1, idx2 = _split_index(index)
    table = jnp.pad(delta, ((0, 0), (0, D - delta.shape[1])))
    xyz_cm, rot_cm = _deform(table, idx0, idx1, idx2)
    return xyz_cm[:, :N].T, rot_cm[:, :N].T


# speedup vs baseline: 34.3236x; 1.0687x over previous
import jax
import jax.numpy as jnp
from jax import lax
from jax.experimental import pallas as pl
from jax.experimental.pallas import tpu as pltpu
from jax.experimental.pallas import tpu_sc as plsc

N = 1_000_000
D = 8
C = 1600
NCH = N // C
NW = 32
GB = [128] * 12 + [64]
GOFF = [sum(GB[:i]) for i in range(len(GB))]
S0, S1, S2 = 59, 29, 14
OFF2 = S0**3 + S1**3
NL2 = S2**3
NPAD = 1_000_064
SPLIT_BLK = 16384


def _rsqrt(x):
    i = plsc.bitcast(x, jnp.int32)
    y = plsc.bitcast(jnp.int32(0x5F3759DF) - (i >> 1), jnp.float32)
    for _ in range(2):
        y = y * (1.5 - 0.5 * x * y * y)
    return y


def _split_body(idx_ref, o0, o1, o2):
    o0[...] = idx_ref[0, :]
    o1[...] = idx_ref[1, :]
    o2[...] = idx_ref[2, :]


def _split_index(index):
    return pl.pallas_call(
        _split_body,
        grid=(pl.cdiv(N, SPLIT_BLK),),
        in_specs=[pl.BlockSpec((3, SPLIT_BLK), lambda i: (0, i))],
        out_specs=[pl.BlockSpec((SPLIT_BLK,), lambda i: (i,))] * 3,
        out_shape=[jax.ShapeDtypeStruct((N,), jnp.int32)] * 3,
    )(index)


def _sc_body(idx0_hbm, idx1_hbm, idx2_hbm, table_hbm, xyz_out, rot_out,
             idx_v, acc_v, l2_v, xyz_v, rot_v,
             sem_l0, sem_l1, sem_o0, sem_o1, sem_o2):
    cid = lax.axis_index("c")
    sid = lax.axis_index("s")
    w = sid * 2 + cid
    nw_chunks = (NCH - w + NW - 1) // NW
    sem_o = [sem_o0, sem_o1, sem_o2]
    idx_l = [idx0_hbm, idx1_hbm, idx2_hbm]

    pltpu.sync_copy(table_hbm.at[pl.ds(OFF2, NL2), :], l2_v)

    def issue_a(k, b):
        base = (w + k * NW) * C
        for l in range(3):
            pltpu.sync_copy(idx_l[l].at[pl.ds(base, C)],
                            idx_v.at[b, pl.ds(l * C, C)])
        for g, off in zip(GB, GOFF):
            pltpu.async_copy(
                table_hbm.at[idx_v.at[b, pl.ds(off, g)]],
                acc_v.at[b, pl.ds(off, g), :],
                sem_l0)

    def drain_l0(b):
        for g, off in zip(GB, GOFF):
            pltpu.make_async_copy(
                table_hbm.at[idx_v.at[b, pl.ds(off, g)]],
                acc_v.at[b, pl.ds(off, g), :],
                sem_l0).wait()

    def issue_b(b):
        for g, off in zip(GB, GOFF):
            pltpu.async_copy(
                table_hbm.at[idx_v.at[b, pl.ds(C + off, g)]],
                acc_v.at[b, pl.ds(off, g), :],
                sem_l1, add=True)

    def drain_l1(b):
        for g, off in zip(GB, GOFF):
            pltpu.make_async_copy(
                table_hbm.at[idx_v.at[b, pl.ds(C + off, g)]],
                acc_v.at[b, pl.ds(off, g), :],
                sem_l1).wait()

    def issue_out(k, b):
        base = (w + k * NW) * C
        pltpu.async_copy(xyz_v.at[b], xyz_out.at[:, pl.ds(base, C)], sem_o[b])
        pltpu.async_copy(rot_v.at[b], rot_out.at[:, pl.ds(base, C)], sem_o[b])

    def drain_out(b):
        pltpu.make_async_copy(xyz_v.at[b], xyz_out.at[:, pl.ds(0, C)],
                              sem_o[b]).wait()
        pltpu.make_async_copy(rot_v.at[b], rot_out.at[:, pl.ds(0, C)],
                              sem_o[b]).wait()

    def norm_chunk(b):
        accb = acc_v.at[b]
        idxb = idx_v.at[b]
        xyzb = xyz_v.at[b]
        rotb = rot_v.at[b]

        def norm16(j, carry2):
            rows = j * 16 + lax.iota(jnp.int32, 16)
            i2 = idxb[pl.ds(2 * C + j * 16, 16)] - OFF2
            cols = []
            for c in range(7):
                fc = jnp.full((16,), c, jnp.int32)
                cols.append(plsc.load_gather(accb, [rows, fc])
                            + plsc.load_gather(l2_v, [i2, fc]))
            q0, q1, q2, q3 = cols[3], cols[4], cols[5], cols[6]
            n2 = q0 * q0 + q1 * q1 + q2 * q2 + q3 * q3
            y = _rsqrt(n2)
            inv = jnp.where(n2 * y >= 1e-12, y, 1e12)
            for c in range(3):
                plsc.store_scatter(xyzb, [jnp.full((16,), c, jnp.int32), rows],
                                   cols[c])
            for k in range(4):
                plsc.store_scatter(rotb, [jnp.full((16,), k, jnp.int32), rows],
                                   cols[3 + k] * inv)
            return carry2

        lax.fori_loop(0, C // 16, norm16, 0)

    issue_a(0, 0)
    issue_a(1, 1)
    drain_l0(0)
    issue_b(0)

    def step(i, bC, bB, bA):
        drain_l1(bC)

        @pl.when(i + 1 < nw_chunks)
        def _():
            drain_l0(bB)
            issue_b(bB)

        @pl.when(i + 2 < nw_chunks)
        def _():
            issue_a(i + 2, bA)

        @pl.when(i >= 3)
        def _():
            drain_out(bC)

        norm_chunk(bC)
        issue_out(i, bC)

    def triple_body(t, carry):
        i = 3 * t
        step(i, 0, 1, 2)

        @pl.when(i + 1 < nw_chunks)
        def _():
            step(i + 1, 1, 2, 0)

        @pl.when(i + 2 < nw_chunks)
        def _():
            step(i + 2, 2, 0, 1)

        return carry

    lax.fori_loop(0, (nw_chunks + 2) // 3, triple_body, 0)
    drain_out(0)
    drain_out(1)
    drain_out(2)


@jax.jit
def _deform(table, idx0, idx1, idx2):
    mesh = plsc.VectorSubcoreMesh(core_axis_name="c", subcore_axis_name="s")
    f = pl.kernel(
        _sc_body,
        out_type=(jax.ShapeDtypeStruct((3, NPAD), jnp.float32),
                  jax.ShapeDtypeStruct((4, NPAD), jnp.float32)),
        mesh=mesh,
        scratch_types=[
            pltpu.VMEM((3, 3 * C), jnp.int32),
            pltpu.VMEM((3, C, D), jnp.float32),
            pltpu.VMEM((NL2, D), jnp.float32),
            pltpu.VMEM((3, 3, C), jnp.float32),
            pltpu.VMEM((3, 4, C), jnp.float32),
            pltpu.SemaphoreType.DMA,
            pltpu.SemaphoreType.DMA,
            pltpu.SemaphoreType.DMA,
            pltpu.SemaphoreType.DMA,
            pltpu.SemaphoreType.DMA,
        ],
        compiler_params=pltpu.CompilerParams(use_tc_tiling_on_sc=False,
                                             needs_layout_passes=False),
    )
    return f(idx0, idx1, idx2, table)


def kernel(xyz, delta, index):
    idx0, idx1, idx2 = _split_index(index)
    table = jnp.pad(delta, ((0, 0), (0, D - delta.shape[1])))
    xyz_cm, rot_cm = _deform(table, idx0, idx1, idx2)
    return xyz_cm[:, :N].T, rot_cm[:, :N].T


# speedup vs baseline: 39.7345x; 1.1576x over previous
import jax
import jax.numpy as jnp
from jax import lax
from jax.experimental import pallas as pl
from jax.experimental.pallas import tpu as pltpu
from jax.experimental.pallas import tpu_sc as plsc

N = 1_000_000
D = 8
C = 1600
NCH = N // C
NW = 32
GB = [128] * 12 + [64]
GOFF = [sum(GB[:i]) for i in range(len(GB))]
S0, S1, S2 = 59, 29, 14
OFF2 = S0**3 + S1**3
NL2 = S2**3
NPAD = 1_000_064
VTAB = S0**3 + S1**3 + S2**3
TBC = 1024
NTB_FULL = VTAB // TBC
SPLIT_BLK = 16384


def _rsqrt(x):
    i = plsc.bitcast(x, jnp.int32)
    y = plsc.bitcast(jnp.int32(0x5F3759DF) - (i >> 1), jnp.float32)
    for _ in range(2):
        y = y * (1.5 - 0.5 * x * y * y)
    return y


def _split_body(idx_ref, o0, o1, o2):
    o0[...] = idx_ref[0, :]
    o1[...] = idx_ref[1, :]
    o2[...] = idx_ref[2, :]


def _split_index(index):
    return pl.pallas_call(
        _split_body,
        grid=(pl.cdiv(N, SPLIT_BLK),),
        in_specs=[pl.BlockSpec((3, SPLIT_BLK), lambda i: (0, i))],
        out_specs=[pl.BlockSpec((SPLIT_BLK,), lambda i: (i,))] * 3,
        out_shape=[jax.ShapeDtypeStruct((N,), jnp.int32)] * 3,
    )(index)


def _sc_body(idx0_hbm, idx1_hbm, idx2_hbm, tabT_hbm, xyz_out, rot_out, table_hbm,
             idx_v, acc_v, l2_v, xyz_v, rot_v, bcol_v, brow_v,
             sem_l0, sem_l1, sem_o0, sem_o1, sem_o2):
    cid = lax.axis_index("c")
    sid = lax.axis_index("s")
    w = sid * 2 + cid
    nw_chunks = (NCH - w + NW - 1) // NW
    sem_o = [sem_o0, sem_o1, sem_o2]
    idx_l = [idx0_hbm, idx1_hbm, idx2_hbm]

    def build_rows(r0, nrows):
        for c in range(7):
            pltpu.sync_copy(tabT_hbm.at[pl.ds(c * VTAB + r0, nrows)],
                            bcol_v.at[pl.ds(c * TBC, nrows)])

        def inter16(j, carry2):
            rows = j * 16 + lax.iota(jnp.int32, 16)
            for c in range(7):
                v = plsc.load_gather(bcol_v, [c * TBC + rows])
                plsc.store_scatter(brow_v,
                                   [rows, jnp.full((16,), c, jnp.int32)], v)
            return carry2

        lax.fori_loop(0, nrows // 16, inter16, 0)
        pltpu.sync_copy(brow_v.at[pl.ds(0, nrows), :],
                        table_hbm.at[pl.ds(r0, nrows), :])

    n_full = (NTB_FULL - sid + 15) // 16

    def build_chunk(k, carry):
        build_rows((sid + k * 16) * TBC, TBC)
        return carry

    lax.fori_loop(0, n_full, build_chunk, 0)

    @pl.when(sid == NTB_FULL % 16)
    def _():
        build_rows(NTB_FULL * TBC, VTAB - NTB_FULL * TBC)

    plsc.subcore_barrier()

    pltpu.sync_copy(table_hbm.at[pl.ds(OFF2, NL2), :], l2_v)

    def issue_a(k, b):
        base = (w + k * NW) * C
        for l in range(3):
            pltpu.sync_copy(idx_l[l].at[pl.ds(base, C)],
                            idx_v.at[b, pl.ds(l * C, C)])
        for g, off in zip(GB, GOFF):
            pltpu.async_copy(
                table_hbm.at[idx_v.at[b, pl.ds(off, g)]],
                acc_v.at[b, pl.ds(off, g), :],
                sem_l0)

    def drain_l0(b):
        for g, off in zip(GB, GOFF):
            pltpu.make_async_copy(
                table_hbm.at[idx_v.at[b, pl.ds(off, g)]],
                acc_v.at[b, pl.ds(off, g), :],
                sem_l0).wait()

    def issue_b(b):
        for g, off in zip(GB, GOFF):
            pltpu.async_copy(
                table_hbm.at[idx_v.at[b, pl.ds(C + off, g)]],
                acc_v.at[b, pl.ds(off, g), :],
                sem_l1, add=True)

    def drain_l1(b):
        for g, off in zip(GB, GOFF):
            pltpu.make_async_copy(
                table_hbm.at[idx_v.at[b, pl.ds(C + off, g)]],
                acc_v.at[b, pl.ds(off, g), :],
                sem_l1).wait()

    def issue_out(k, b):
        base = (w + k * NW) * C
        pltpu.async_copy(xyz_v.at[b], xyz_out.at[:, pl.ds(base, C)], sem_o[b])
        pltpu.async_copy(rot_v.at[b], rot_out.at[:, pl.ds(base, C)], sem_o[b])

    def drain_out(b):
        pltpu.make_async_copy(xyz_v.at[b], xyz_out.at[:, pl.ds(0, C)],
                              sem_o[b]).wait()
        pltpu.make_async_copy(rot_v.at[b], rot_out.at[:, pl.ds(0, C)],
                              sem_o[b]).wait()

    def norm_chunk(b):
        accb = acc_v.at[b]
        idxb = idx_v.at[b]
        xyzb = xyz_v.at[b]
        rotb = rot_v.at[b]

        def norm16(j, carry2):
            rows = j * 16 + lax.iota(jnp.int32, 16)
            i2 = idxb[pl.ds(2 * C + j * 16, 16)] - OFF2
            cols = []
            for c in range(7):
                fc = jnp.full((16,), c, jnp.int32)
                cols.append(plsc.load_gather(accb, [rows, fc])
                            + plsc.load_gather(l2_v, [i2, fc]))
            q0, q1, q2, q3 = cols[3], cols[4], cols[5], cols[6]
            n2 = q0 * q0 + q1 * q1 + q2 * q2 + q3 * q3
            y = _rsqrt(n2)
            inv = jnp.where(n2 * y >= 1e-12, y, 1e12)
            for c in range(3):
                plsc.store_scatter(xyzb, [jnp.full((16,), c, jnp.int32), rows],
                                   cols[c])
            for k in range(4):
                plsc.store_scatter(rotb, [jnp.full((16,), k, jnp.int32), rows],
                                   cols[3 + k] * inv)
            return carry2

        lax.fori_loop(0, C // 16, norm16, 0)

    issue_a(0, 0)
    issue_a(1, 1)
    drain_l0(0)
    issue_b(0)

    def step(i, bC, bB, bA):
        drain_l1(bC)

        @pl.when(i + 1 < nw_chunks)
        def _():
            drain_l0(bB)
            issue_b(bB)

        @pl.when(i + 2 < nw_chunks)
        def _():
            issue_a(i + 2, bA)

        @pl.when(i >= 3)
        def _():
            drain_out(bC)

        norm_chunk(bC)
        issue_out(i, bC)

    def triple_body(t, carry):
        i = 3 * t
        step(i, 0, 1, 2)

        @pl.when(i + 1 < nw_chunks)
        def _():
            step(i + 1, 1, 2, 0)

        @pl.when(i + 2 < nw_chunks)
        def _():
            step(i + 2, 2, 0, 1)

        return carry

    lax.fori_loop(0, (nw_chunks + 2) // 3, triple_body, 0)
    drain_out(0)
    drain_out(1)
    drain_out(2)


@jax.jit
def _deform(tabT, idx0, idx1, idx2):
    mesh = plsc.VectorSubcoreMesh(core_axis_name="c", subcore_axis_name="s")
    f = pl.kernel(
        _sc_body,
        out_type=(jax.ShapeDtypeStruct((3, NPAD), jnp.float32),
                  jax.ShapeDtypeStruct((4, NPAD), jnp.float32),
                  jax.ShapeDtypeStruct((VTAB, D), jnp.float32)),
        mesh=mesh,
        scratch_types=[
            pltpu.VMEM((3, 3 * C), jnp.int32),
            pltpu.VMEM((3, C, D), jnp.float32),
            pltpu.VMEM((NL2, D), jnp.float32),
            pltpu.VMEM((3, 3, C), jnp.float32),
            pltpu.VMEM((3, 4, C), jnp.float32),
            pltpu.VMEM((7 * TBC,), jnp.float32),
            pltpu.VMEM((TBC, D), jnp.float32),
            pltpu.SemaphoreType.DMA,
            pltpu.SemaphoreType.DMA,
            pltpu.SemaphoreType.DMA,
            pltpu.SemaphoreType.DMA,
            pltpu.SemaphoreType.DMA,
        ],
        compiler_params=pltpu.CompilerParams(use_tc_tiling_on_sc=False,
                                             needs_layout_passes=False),
    )
    xyz_cm, rot_cm, _ = f(idx0, idx1, idx2, tabT)
    return xyz_cm, rot_cm


def kernel(xyz, delta, index):
    idx0, idx1, idx2 = _split_index(index)
    xyz_cm, rot_cm = _deform(delta.T.reshape(-1), idx0, idx1, idx2)
    return xyz_cm[:, :N].T, rot_cm[:, :N].T


# speedup vs baseline: 43.9711x; 1.1066x over previous
import jax
import jax.numpy as jnp
from jax import lax
from jax.experimental import pallas as pl
from jax.experimental.pallas import tpu as pltpu
from jax.experimental.pallas import tpu_sc as plsc

N = 1_000_000
D = 8
C = 1600
NCH = N // C
NW = 32
GB = [128] * 12 + [64]
GOFF = [sum(GB[:i]) for i in range(len(GB))]
S0, S1, S2 = 59, 29, 14
OFF2 = S0**3 + S1**3
NL2 = S2**3
NPAD = 1_000_064
VTAB = S0**3 + S1**3 + S2**3
TBC = 1024
NTB_FULL = VTAB // TBC
SPLIT_BLK = 16384


def _rsqrt(x):
    i = plsc.bitcast(x, jnp.int32)
    y = plsc.bitcast(jnp.int32(0x5F3759DF) - (i >> 1), jnp.float32)
    for _ in range(2):
        y = y * (1.5 - 0.5 * x * y * y)
    return y


def _split_body(idx_ref, o0, o1, o2):
    o0[...] = idx_ref[0, :]
    o1[...] = idx_ref[1, :]
    o2[...] = idx_ref[2, :]


def _split_index(index):
    return pl.pallas_call(
        _split_body,
        grid=(pl.cdiv(N, SPLIT_BLK),),
        in_specs=[pl.BlockSpec((3, SPLIT_BLK), lambda i: (0, i))],
        out_specs=[pl.BlockSpec((SPLIT_BLK,), lambda i: (i,))] * 3,
        out_shape=[jax.ShapeDtypeStruct((N,), jnp.int32)] * 3,
    )(index)


def _sc_body(idx0_hbm, idx1_hbm, idx2_hbm, tabT_hbm, xyz_out, rot_out, table_hbm,
             idx_v, acc_v, l2_v, xyz_v, rot_v, bcol_v, brow_v,
             sem_l0, sem_l1, sem_o0, sem_o1, sem_o2):
    cid = lax.axis_index("c")
    sid = lax.axis_index("s")
    w = sid * 2 + cid
    nw_chunks = (NCH - w + NW - 1) // NW
    sem_o = [sem_o0, sem_o1, sem_o2]
    idx_l = [idx0_hbm, idx1_hbm, idx2_hbm]

    def build_rows(r0, nrows):
        for c in range(7):
            pltpu.sync_copy(tabT_hbm.at[pl.ds(c * VTAB + r0, nrows)],
                            bcol_v.at[pl.ds(c * TBC, nrows)])

        def inter16(j, carry2):
            rows = j * 16 + lax.iota(jnp.int32, 16)
            for c in range(7):
                v = plsc.load_gather(bcol_v, [c * TBC + rows])
                plsc.store_scatter(brow_v,
                                   [rows, jnp.full((16,), c, jnp.int32)], v)
            return carry2

        lax.fori_loop(0, nrows // 16, inter16, 0)
        pltpu.sync_copy(brow_v.at[pl.ds(0, nrows), :],
                        table_hbm.at[pl.ds(r0, nrows), :])

    n_full = (NTB_FULL - sid + 15) // 16

    def build_chunk(k, carry):
        build_rows((sid + k * 16) * TBC, TBC)
        return carry

    lax.fori_loop(0, n_full, build_chunk, 0)

    @pl.when(sid == NTB_FULL % 16)
    def _():
        build_rows(NTB_FULL * TBC, VTAB - NTB_FULL * TBC)

    plsc.subcore_barrier()

    pltpu.sync_copy(table_hbm.at[pl.ds(OFF2, NL2), :], l2_v)

    def issue_a(k, b):
        base = (w + k * NW) * C
        for l in range(3):
            pltpu.sync_copy(idx_l[l].at[pl.ds(base, C)],
                            idx_v.at[b, pl.ds(l * C, C)])
        for g, off in zip(GB, GOFF):
            pltpu.async_copy(
                table_hbm.at[idx_v.at[b, pl.ds(off, g)]],
                acc_v.at[b, pl.ds(off, g), :],
                sem_l0)

    def drain_l0(b):
        for g, off in zip(GB, GOFF):
            pltpu.make_async_copy(
                table_hbm.at[idx_v.at[b, pl.ds(off, g)]],
                acc_v.at[b, pl.ds(off, g), :],
                sem_l0).wait()

    def issue_b(b):
        for g, off in zip(GB, GOFF):
            pltpu.async_copy(
                table_hbm.at[idx_v.at[b, pl.ds(C + off, g)]],
                acc_v.at[b, pl.ds(off, g), :],
                sem_l1, add=True)

    def drain_l1(b):
        for g, off in zip(GB, GOFF):
            pltpu.make_async_copy(
                table_hbm.at[idx_v.at[b, pl.ds(C + off, g)]],
                acc_v.at[b, pl.ds(off, g), :],
                sem_l1).wait()

    def issue_out(k, b):
        base = (w + k * NW) * C
        pltpu.async_copy(xyz_v.at[b], xyz_out.at[:, pl.ds(base, C)], sem_o[b])
        pltpu.async_copy(rot_v.at[b], rot_out.at[:, pl.ds(base, C)], sem_o[b])

    def drain_out(b):
        pltpu.make_async_copy(xyz_v.at[b], xyz_out.at[:, pl.ds(0, C)],
                              sem_o[b]).wait()
        pltpu.make_async_copy(rot_v.at[b], rot_out.at[:, pl.ds(0, C)],
                              sem_o[b]).wait()

    def norm_chunk(b):
        accb = acc_v.at[b]
        idxb = idx_v.at[b]
        xyzb = xyz_v.at[b]
        rotb = rot_v.at[b]

        @plsc.parallel_loop(0, C // 16, unroll=2)
        def norm16(j):
            rows = j * 16 + lax.iota(jnp.int32, 16)
            i2 = idxb[pl.ds(2 * C + j * 16, 16)] - OFF2
            cols = []
            for c in range(7):
                fc = jnp.full((16,), c, jnp.int32)
                cols.append(plsc.load_gather(accb, [rows, fc])
                            + plsc.load_gather(l2_v, [i2, fc]))
            q0, q1, q2, q3 = cols[3], cols[4], cols[5], cols[6]
            n2 = q0 * q0 + q1 * q1 + q2 * q2 + q3 * q3
            y = _rsqrt(n2)
            inv = jnp.where(n2 * y >= 1e-12, y, 1e12)
            for c in range(3):
                plsc.store_scatter(xyzb, [jnp.full((16,), c, jnp.int32), rows],
                                   cols[c])
            for k in range(4):
                plsc.store_scatter(rotb, [jnp.full((16,), k, jnp.int32), rows],
                                   cols[3 + k] * inv)

    issue_a(0, 0)
    issue_a(1, 1)
    drain_l0(0)
    issue_b(0)

    def step(i, bC, bB, bA):
        drain_l1(bC)

        @pl.when(i + 1 < nw_chunks)
        def _():
            drain_l0(bB)
            issue_b(bB)

        @pl.when(i + 2 < nw_chunks)
        def _():
            issue_a(i + 2, bA)

        @pl.when(i >= 3)
        def _():
            drain_out(bC)

        norm_chunk(bC)
        issue_out(i, bC)

    def triple_body(t, carry):
        i = 3 * t
        step(i, 0, 1, 2)

        @pl.when(i + 1 < nw_chunks)
        def _():
            step(i + 1, 1, 2, 0)

        @pl.when(i + 2 < nw_chunks)
        def _():
            step(i + 2, 2, 0, 1)

        return carry

    lax.fori_loop(0, (nw_chunks + 2) // 3, triple_body, 0)
    drain_out(0)
    drain_out(1)
    drain_out(2)


@jax.jit
def _deform(tabT, idx0, idx1, idx2):
    mesh = plsc.VectorSubcoreMesh(core_axis_name="c", subcore_axis_name="s")
    f = pl.kernel(
        _sc_body,
        out_type=(jax.ShapeDtypeStruct((3, NPAD), jnp.float32),
                  jax.ShapeDtypeStruct((4, NPAD), jnp.float32),
                  jax.ShapeDtypeStruct((VTAB, D), jnp.float32)),
        mesh=mesh,
        scratch_types=[
            pltpu.VMEM((3, 3 * C), jnp.int32),
            pltpu.VMEM((3, C, D), jnp.float32),
            pltpu.VMEM((NL2, D), jnp.float32),
            pltpu.VMEM((3, 3, C), jnp.float32),
            pltpu.VMEM((3, 4, C), jnp.float32),
            pltpu.VMEM((7 * TBC,), jnp.float32),
            pltpu.VMEM((TBC, D), jnp.float32),
            pltpu.SemaphoreType.DMA,
            pltpu.SemaphoreType.DMA,
            pltpu.SemaphoreType.DMA,
            pltpu.SemaphoreType.DMA,
            pltpu.SemaphoreType.DMA,
        ],
        compiler_params=pltpu.CompilerParams(use_tc_tiling_on_sc=False,
                                             needs_layout_passes=False),
    )
    xyz_cm, rot_cm, _ = f(idx0, idx1, idx2, tabT)
    return xyz_cm, rot_cm


def kernel(xyz, delta, index):
    idx0, idx1, idx2 = _split_index(index)
    xyz_cm, rot_cm = _deform(delta.T.reshape(-1), idx0, idx1, idx2)
    return xyz_cm[:, :N].T, rot_cm[:, :N].T
